# baseline (speedup 1.0000x reference)
import functools

import jax
import jax.numpy as jnp
from jax import lax
from jax.experimental import pallas as pl
from jax.experimental.pallas import tpu as pltpu
from jax.experimental.pallas import tpu_sc as plsc

HIDDEN = 768
PAD_IDX = 1
EPS = 1e-05

_NC = 2
_NS = 16
_NW = _NC * _NS


def _lane_sum(v, idx16):
    for k in (1, 2, 4, 8):
        v = v + v.at[idx16 ^ k].get(mode="promise_in_bounds")
    return v


def _sc_gather_sum(word_emb, pos_emb, ids2d):
    nrow, seq_len = ids2d.shape
    tok = nrow * seq_len
    tpw = tok // _NW
    wpr = seq_len // tpw
    ch = 16
    nchunk = tpw // ch
    mesh = plsc.VectorSubcoreMesh(core_axis_name="c", subcore_axis_name="s")
    buf = pltpu.VMEM((ch, HIDDEN), jnp.float32)

    @functools.partial(
        pl.kernel,
        out_type=[jax.ShapeDtypeStruct((tok, HIDDEN), jnp.float32),
                  jax.ShapeDtypeStruct((nrow, seq_len), jnp.int32)],
        mesh=mesh,
        scratch_types=[
            pltpu.VMEM((seq_len,), jnp.int32),
            pltpu.VMEM((tpw,), jnp.int32),
            pltpu.VMEM((16,), jnp.int32),
            buf, buf, buf, buf, buf, buf,
            pltpu.SemaphoreType.DMA, pltpu.SemaphoreType.DMA,
            pltpu.SemaphoreType.DMA, pltpu.SemaphoreType.DMA,
            pltpu.SemaphoreType.DMA, pltpu.SemaphoreType.DMA,
            pltpu.SemaphoreType.DMA,
        ],
    )
    def k(word_hbm, pos_hbm, ids_hbm, out_hbm, pid_hbm, ids_row, pid_v,
          acc_ref, w0, w1, p0, p1, o0, o1, ws0, ws1, ps0, ps1, os0, os1,
          psem_out):
        wb, pb, ob = [w0, w1], [p0, p1], [o0, o1]
        wsem, psem, osem = [ws0, ws1], [ps0, ps1], [os0, os1]
        wid = lax.axis_index("s") * _NC + lax.axis_index("c")
        base = wid * tpw
        row = wid // wpr
        pre = (wid % wpr) * tpw
        pltpu.sync_copy(ids_hbm.at[row], ids_row)
        idx16 = lax.iota(jnp.int32, 16)
        last16 = idx16 * 0 + 15

        acc_ref[...] = jnp.zeros((16,), jnp.int32)

        @pl.loop(0, seq_len, step=16)
        def _prefix(i):
            m = jnp.minimum(jnp.abs(ids_row[pl.ds(i, 16)] - PAD_IDX), 1)
            w = jnp.minimum(jnp.maximum(pre - i, 0), 1)
            acc_ref[...] = acc_ref[...] + m * w

        carry = _lane_sum(acc_ref[...], idx16)
        scan_masks = [jnp.minimum(jnp.maximum(idx16 - (kk - 1), 0), 1)
                      for kk in (1, 2, 4, 8)]
        for j in range(0, tpw, 16):
            v = ids_row[pl.ds(pre + j, 16)]
            m = jnp.minimum(jnp.abs(v - PAD_IDX), 1)
            c = m
            for kk, sm in zip((1, 2, 4, 8), scan_masks):
                shifted = c.at[jnp.maximum(idx16 - kk, 0)].get(
                    mode="promise_in_bounds")
                c = c + shifted * sm
            pid_v[pl.ds(j, 16)] = (c + carry) * m + PAD_IDX
            carry = carry + c.at[last16].get(mode="promise_in_bounds")

        pltpu.async_copy(pid_v, pid_hbm.at[row, pl.ds(pre, tpw)], psem_out)

        def issue_gathers(i, b):
            pltpu.async_copy(
                word_hbm.at[ids_row.at[pl.ds(pre + i * ch, ch)]], wb[b],
                wsem[b])
            pltpu.async_copy(
                pos_hbm.at[pid_v.at[pl.ds(i * ch, ch)]], pb[b], psem[b])

        def wait_gathers(b):
            pltpu.make_async_copy(
                word_hbm.at[ids_row.at[pl.ds(0, ch)]], wb[b],
                wsem[b]).wait()
            pltpu.make_async_copy(
                pos_hbm.at[pid_v.at[pl.ds(0, ch)]], pb[b], psem[b]).wait()

        def wait_owrite(b):
            pltpu.make_async_copy(
                ob[b], out_hbm.at[pl.ds(base, ch)], osem[b]).wait()

        issue_gathers(0, 0)
        issue_gathers(1, 1)

        @pl.loop(0, nchunk, step=2)
        def _pair(g):
            for b in range(2):
                wait_gathers(b)

                @pl.when(g >= 2 - b)
                def _():
                    wait_owrite(b)

                @pl.loop(0, ch)
                def _row(r):
                    for c in range(0, HIDDEN, 16):
                        sl = (r, pl.ds(c, 16))
                        ob[b][sl] = wb[b][sl] + pb[b][sl]

                pltpu.async_copy(
                    ob[b], out_hbm.at[pl.ds(base + (g + b) * ch, ch)],
                    osem[b])

                @pl.when(g < nchunk - 2 - b)
                def _():
                    issue_gathers(g + b + 2, b)

        wait_owrite(0)
        wait_owrite(1)
        pltpu.make_async_copy(pid_v, pid_hbm.at[row, pl.ds(pre, tpw)],
                              psem_out).wait()

    return k(word_emb, pos_emb, ids2d)


def _ln_body3(x_ref, tok_ref, w_ref, b_ref, o_ref):
    x = x_ref[...] + tok_ref[...]
    mean = jnp.mean(x, axis=-1, keepdims=True)
    xc = x - mean
    var = jnp.mean(xc * xc, axis=-1, keepdims=True)
    o_ref[0] = xc * lax.rsqrt(var + EPS) * w_ref[...] + b_ref[...]


def _ln(summed, tok_row, ln_w, ln_b, nrow, seq_len):
    blk = 2048
    spb = seq_len // blk
    return pl.pallas_call(
        _ln_body3,
        grid=(nrow, spb),
        in_specs=[
            pl.BlockSpec((blk, HIDDEN), lambda i, j: (i * spb + j, 0)),
            pl.BlockSpec((1, HIDDEN), lambda i, j: (0, 0)),
            pl.BlockSpec((1, HIDDEN), lambda i, j: (0, 0)),
            pl.BlockSpec((1, HIDDEN), lambda i, j: (0, 0)),
        ],
        out_specs=pl.BlockSpec((1, blk, HIDDEN), lambda i, j: (i, j, 0)),
        out_shape=jax.ShapeDtypeStruct((nrow, seq_len, HIDDEN),
                                       jnp.float32),
    )(summed, tok_row, ln_w, ln_b)


def kernel(input_ids, word_emb, pos_emb, tok_emb, ln_w, ln_b):
    b, s = input_ids.shape
    summed, pid = _sc_gather_sum(word_emb, pos_emb, input_ids)
    out = _ln(summed, tok_emb[0:1], ln_w.reshape(1, HIDDEN),
              ln_b.reshape(1, HIDDEN), b, s)
    return out, pid


# speedup vs baseline: 1.0121x; 1.0121x over previous
import functools

import jax
import jax.numpy as jnp
from jax import lax
from jax.experimental import pallas as pl
from jax.experimental.pallas import tpu as pltpu
from jax.experimental.pallas import tpu_sc as plsc

HIDDEN = 768
PAD_IDX = 1
EPS = 1e-05

_NC = 2
_NS = 16
_NW = _NC * _NS


def _lane_sum(v, idx16):
    for k in (1, 2, 4, 8):
        v = v + v.at[idx16 ^ k].get(mode="promise_in_bounds")
    return v


def _sc_gather_sum(word_emb, pos_emb, ids2d):
    nrow, seq_len = ids2d.shape
    tok = nrow * seq_len
    tpw = tok // _NW
    wpr = seq_len // tpw
    ch = 16
    nchunk = tpw // ch
    mesh = plsc.VectorSubcoreMesh(core_axis_name="c", subcore_axis_name="s")
    buf = pltpu.VMEM((ch, HIDDEN), jnp.float32)

    @functools.partial(
        pl.kernel,
        out_type=[jax.ShapeDtypeStruct((tok, HIDDEN), jnp.float32),
                  jax.ShapeDtypeStruct((nrow, seq_len), jnp.int32)],
        mesh=mesh,
        scratch_types=[
            pltpu.VMEM((seq_len,), jnp.int32),
            pltpu.VMEM((tpw,), jnp.int32),
            pltpu.VMEM((16,), jnp.int32),
            buf, buf, buf, buf, buf, buf,
            pltpu.SemaphoreType.DMA, pltpu.SemaphoreType.DMA,
            pltpu.SemaphoreType.DMA, pltpu.SemaphoreType.DMA,
            pltpu.SemaphoreType.DMA, pltpu.SemaphoreType.DMA,
            pltpu.SemaphoreType.DMA,
        ],
    )
    def k(word_hbm, pos_hbm, ids_hbm, out_hbm, pid_hbm, ids_row, pid_v,
          acc_ref, w0, w1, p0, p1, o0, o1, ws0, ws1, ps0, ps1, os0, os1,
          psem_out):
        wb, pb, ob = [w0, w1], [p0, p1], [o0, o1]
        wsem, psem, osem = [ws0, ws1], [ps0, ps1], [os0, os1]
        wid = lax.axis_index("s") * _NC + lax.axis_index("c")
        base = wid * tpw
        row = wid // wpr
        pre = (wid % wpr) * tpw
        pltpu.sync_copy(ids_hbm.at[row], ids_row)
        idx16 = lax.iota(jnp.int32, 16)
        last16 = idx16 * 0 + 15

        def issue_word(i, b):
            pltpu.async_copy(
                word_hbm.at[ids_row.at[pl.ds(pre + i * ch, ch)]], wb[b],
                wsem[b])

        def issue_pos(i, b):
            pltpu.async_copy(
                pos_hbm.at[pid_v.at[pl.ds(i * ch, ch)]], pb[b], psem[b])

        issue_word(0, 0)
        issue_word(1, 1)

        acc_ref[...] = jnp.zeros((16,), jnp.int32)

        @pl.loop(0, seq_len, step=16)
        def _prefix(i):
            m = jnp.minimum(jnp.abs(ids_row[pl.ds(i, 16)] - PAD_IDX), 1)
            w = jnp.minimum(jnp.maximum(pre - i, 0), 1)
            acc_ref[...] = acc_ref[...] + m * w

        carry = _lane_sum(acc_ref[...], idx16)
        scan_masks = [jnp.minimum(jnp.maximum(idx16 - (kk - 1), 0), 1)
                      for kk in (1, 2, 4, 8)]
        for j in range(0, tpw, 16):
            v = ids_row[pl.ds(pre + j, 16)]
            m = jnp.minimum(jnp.abs(v - PAD_IDX), 1)
            c = m
            for kk, sm in zip((1, 2, 4, 8), scan_masks):
                shifted = c.at[jnp.maximum(idx16 - kk, 0)].get(
                    mode="promise_in_bounds")
                c = c + shifted * sm
            pid_v[pl.ds(j, 16)] = (c + carry) * m + PAD_IDX
            carry = carry + c.at[last16].get(mode="promise_in_bounds")

        pltpu.async_copy(pid_v, pid_hbm.at[row, pl.ds(pre, tpw)], psem_out)

        def issue_gathers(i, b):
            issue_word(i, b)
            issue_pos(i, b)

        def wait_gathers(b):
            pltpu.make_async_copy(
                word_hbm.at[ids_row.at[pl.ds(0, ch)]], wb[b],
                wsem[b]).wait()
            pltpu.make_async_copy(
                pos_hbm.at[pid_v.at[pl.ds(0, ch)]], pb[b], psem[b]).wait()

        def wait_owrite(b):
            pltpu.make_async_copy(
                ob[b], out_hbm.at[pl.ds(base, ch)], osem[b]).wait()

        issue_pos(0, 0)
        issue_pos(1, 1)

        @pl.loop(0, nchunk, step=2)
        def _pair(g):
            for b in range(2):
                wait_gathers(b)

                @pl.when(g >= 2 - b)
                def _():
                    wait_owrite(b)

                @pl.loop(0, ch)
                def _row(r):
                    for c in range(0, HIDDEN, 16):
                        sl = (r, pl.ds(c, 16))
                        ob[b][sl] = wb[b][sl] + pb[b][sl]

                pltpu.async_copy(
                    ob[b], out_hbm.at[pl.ds(base + (g + b) * ch, ch)],
                    osem[b])

                @pl.when(g < nchunk - 2 - b)
                def _():
                    issue_gathers(g + b + 2, b)

        wait_owrite(0)
        wait_owrite(1)
        pltpu.make_async_copy(pid_v, pid_hbm.at[row, pl.ds(pre, tpw)],
                              psem_out).wait()

    return k(word_emb, pos_emb, ids2d)


def _ln_body3(x_ref, tok_ref, w_ref, b_ref, o_ref):
    x = x_ref[...] + tok_ref[...]
    mean = jnp.mean(x, axis=-1, keepdims=True)
    xc = x - mean
    var = jnp.mean(xc * xc, axis=-1, keepdims=True)
    o_ref[0] = xc * lax.rsqrt(var + EPS) * w_ref[...] + b_ref[...]


def _ln(summed, tok_row, ln_w, ln_b, nrow, seq_len):
    blk = 2048
    spb = seq_len // blk
    return pl.pallas_call(
        _ln_body3,
        grid=(nrow, spb),
        in_specs=[
            pl.BlockSpec((blk, HIDDEN), lambda i, j: (i * spb + j, 0)),
            pl.BlockSpec((1, HIDDEN), lambda i, j: (0, 0)),
            pl.BlockSpec((1, HIDDEN), lambda i, j: (0, 0)),
            pl.BlockSpec((1, HIDDEN), lambda i, j: (0, 0)),
        ],
        out_specs=pl.BlockSpec((1, blk, HIDDEN), lambda i, j: (i, j, 0)),
        out_shape=jax.ShapeDtypeStruct((nrow, seq_len, HIDDEN),
                                       jnp.float32),
    )(summed, tok_row, ln_w, ln_b)


def kernel(input_ids, word_emb, pos_emb, tok_emb, ln_w, ln_b):
    b, s = input_ids.shape
    summed, pid = _sc_gather_sum(word_emb, pos_emb, input_ids)
    out = _ln(summed, tok_emb[0:1], ln_w.reshape(1, HIDDEN),
              ln_b.reshape(1, HIDDEN), b, s)
    return out, pid
